# baseline (speedup 1.0000x reference)
import functools

import jax
import jax.numpy as jnp
from jax import lax
from jax.experimental import pallas as pl
from jax.experimental.pallas import tpu as pltpu
from jax.experimental.pallas import tpu_sc as plsc

_NUM_WORKERS = 32
_LANES = 16


def _softplus_vec(p):
    u = jnp.exp(-jnp.abs(p))
    w = 1.0 + u
    y = 2.0 * u / (2.0 + u)
    y = y + w * jnp.exp(-y) - 1.0
    y = y + w * jnp.exp(-y) - 1.0
    y = y + w * jnp.exp(-y) - 1.0
    return jnp.maximum(p, 0.0) + y


@functools.lru_cache(maxsize=None)
def _sc_call(n, chunk):
    per_worker = n // _NUM_WORKERS
    n_chunks = per_worker // chunk
    mesh = plsc.VectorSubcoreMesh(core_axis_name="c", subcore_axis_name="s")

    @functools.partial(
        pl.kernel,
        out_type=jax.ShapeDtypeStruct((n,), jnp.float32),
        mesh=mesh,
        compiler_params=pltpu.CompilerParams(needs_layout_passes=False),
        scratch_types=[
            pltpu.VMEM((32,), jnp.float32),
            pltpu.VMEM((32,), jnp.float32),
            pltpu.VMEM((32,), jnp.float32),
            pltpu.VMEM((chunk,), jnp.float32),
            pltpu.VMEM((chunk,), jnp.int32),
            pltpu.VMEM((chunk,), jnp.float32),
        ],
    )
    def body(x_hbm, h_hbm, pat_hbm, zl_hbm, out_hbm, pat_v, zl_v, tab_v, x_v, h_v, o_v):
        wid = lax.axis_index("s") * 2 + lax.axis_index("c")
        base = wid * per_worker

        pltpu.sync_copy(pat_hbm, pat_v)
        pltpu.sync_copy(zl_hbm, zl_v)
        for j in range(2):
            sl = pl.ds(j * _LANES, _LANES)
            p = pat_v[sl]
            z = zl_v[sl]
            sp = _softplus_vec(p)
            one_minus_sig = 1.0 / (1.0 + jnp.exp(z))
            tab_v[sl] = sp * one_minus_sig

        def chunk_body(c, carry):
            off = base + c * chunk
            pltpu.sync_copy(x_hbm.at[pl.ds(off, chunk)], x_v)
            pltpu.sync_copy(h_hbm.at[pl.ds(off, chunk)], h_v)

            def inner(i, carry2):
                s = pl.ds(i * _LANES, _LANES)
                mv = plsc.load_gather(tab_v, [h_v[s]])
                o_v[s] = x_v[s] * mv
                return carry2

            lax.fori_loop(0, chunk // _LANES, inner, 0, unroll=4)
            pltpu.sync_copy(o_v, out_hbm.at[pl.ds(off, chunk)])
            return carry

        lax.fori_loop(0, n_chunks, chunk_body, 0)

    return body


def kernel(x, hour_indices, pattern, zero_logits):
    b, t = x.shape
    n = b * t
    xf = x.reshape(n)
    hf = hour_indices.reshape(n).astype(jnp.int32)
    pat = jnp.pad(pattern.astype(jnp.float32), (0, 8))
    zl = jnp.pad(zero_logits.astype(jnp.float32), (0, 8))
    out = _sc_call(n, 10240)(xf, hf, pat, zl)
    return out.reshape(b, t)


# speedup vs baseline: 1.3952x; 1.3952x over previous
import functools

import jax
import jax.numpy as jnp
from jax import lax
from jax.experimental import pallas as pl
from jax.experimental.pallas import tpu as pltpu
from jax.experimental.pallas import tpu_sc as plsc

_NUM_WORKERS = 32
_LANES = 16


def _softplus_vec(p):
    u = jnp.exp(-jnp.abs(p))
    w = 1.0 + u
    y = 2.0 * u / (2.0 + u)
    y = y + w * jnp.exp(-y) - 1.0
    y = y + w * jnp.exp(-y) - 1.0
    y = y + w * jnp.exp(-y) - 1.0
    return jnp.maximum(p, 0.0) + y


@functools.lru_cache(maxsize=None)
def _sc_call(n, chunk):
    per_worker = n // _NUM_WORKERS
    n_chunks = per_worker // chunk
    assert per_worker % chunk == 0
    mesh = plsc.VectorSubcoreMesh(core_axis_name="c", subcore_axis_name="s")

    @functools.partial(
        pl.kernel,
        out_type=jax.ShapeDtypeStruct((n,), jnp.float32),
        mesh=mesh,
        compiler_params=pltpu.CompilerParams(needs_layout_passes=False),
        scratch_types=[
            pltpu.VMEM((32,), jnp.float32),
            pltpu.VMEM((32,), jnp.float32),
            pltpu.VMEM((32,), jnp.float32),
            pltpu.VMEM((2, chunk), jnp.float32),
            pltpu.VMEM((2, chunk), jnp.int32),
            pltpu.VMEM((2, chunk), jnp.float32),
            pltpu.SemaphoreType.DMA,
            pltpu.SemaphoreType.DMA,
            pltpu.SemaphoreType.DMA,
            pltpu.SemaphoreType.DMA,
            pltpu.SemaphoreType.DMA,
            pltpu.SemaphoreType.DMA,
        ],
    )
    def body(x_hbm, h_hbm, pat_hbm, zl_hbm, out_hbm,
             pat_v, zl_v, tab_v, x_v, h_v, o_v,
             sx0, sx1, sh0, sh1, so0, so1):
        wid = lax.axis_index("s") * 2 + lax.axis_index("c")
        base = wid * per_worker
        sx = (sx0, sx1)
        sh = (sh0, sh1)
        so = (so0, so1)

        pltpu.sync_copy(pat_hbm, pat_v)
        pltpu.sync_copy(zl_hbm, zl_v)
        for j in range(2):
            sl = pl.ds(j * _LANES, _LANES)
            p = pat_v[sl]
            z = zl_v[sl]
            sp = _softplus_vec(p)
            one_minus_sig = 1.0 / (1.0 + jnp.exp(z))
            tab_v[sl] = sp * one_minus_sig

        def start_in(c):
            b = c % 2
            off = base + c * chunk
            cx = pltpu.make_async_copy(x_hbm.at[pl.ds(off, chunk)], x_v.at[b], sx[b])
            cx.start()
            chh = pltpu.make_async_copy(h_hbm.at[pl.ds(off, chunk)], h_v.at[b], sh[b])
            chh.start()
            return cx, chh

        in_copies = [None, None]
        out_copies = [None, None]
        in_copies[0] = start_in(0)
        for c in range(n_chunks):
            b = c % 2
            if c + 1 < n_chunks:
                in_copies[(c + 1) % 2] = start_in(c + 1)
            cx, chh = in_copies[b]
            cx.wait()
            chh.wait()
            if out_copies[b] is not None:
                out_copies[b].wait()

            @plsc.parallel_loop(0, chunk, step=_LANES, unroll=8)
            def _(s):
                sl = pl.ds(s, _LANES)
                mv = plsc.load_gather(tab_v, [h_v[b, sl]])
                o_v[b, sl] = x_v[b, sl] * mv

            co = pltpu.make_async_copy(
                o_v.at[b], out_hbm.at[pl.ds(base + c * chunk, chunk)], so[b])
            co.start()
            out_copies[b] = co
        for b in range(2):
            if out_copies[b] is not None:
                out_copies[b].wait()

    return body


def kernel(x, hour_indices, pattern, zero_logits):
    b, t = x.shape
    n = b * t
    xf = x.reshape(n)
    hf = hour_indices.reshape(n).astype(jnp.int32)
    pat = jnp.pad(pattern.astype(jnp.float32), (0, 8))
    zl = jnp.pad(zero_logits.astype(jnp.float32), (0, 8))
    out = _sc_call(n, 10240)(xf, hf, pat, zl)
    return out.reshape(b, t)


# speedup vs baseline: 2.3798x; 1.7057x over previous
import functools

import jax
import jax.numpy as jnp
from jax import lax
from jax.experimental import pallas as pl
from jax.experimental.pallas import tpu as pltpu
from jax.experimental.pallas import tpu_sc as plsc

_NUM_WORKERS = 32
_LANES = 16


def _softplus_vec(p):
    u = jnp.exp(-jnp.abs(p))
    w = 1.0 + u
    y = 2.0 * u / (2.0 + u)
    y = y + w * jnp.exp(-y) - 1.0
    y = y + w * jnp.exp(-y) - 1.0
    y = y + w * jnp.exp(-y) - 1.0
    return jnp.maximum(p, 0.0) + y


@functools.lru_cache(maxsize=None)
def _sc_call(nrows, ncols, rows_per_chunk):
    per_worker = nrows // _NUM_WORKERS
    n_chunks = per_worker // rows_per_chunk
    assert nrows % _NUM_WORKERS == 0 and per_worker % rows_per_chunk == 0
    col_starts = list(range(0, ncols - _LANES + 1, _LANES))
    if col_starts[-1] + _LANES < ncols:
        col_starts.append(ncols - _LANES)
    mesh = plsc.VectorSubcoreMesh(core_axis_name="c", subcore_axis_name="s")

    @functools.partial(
        pl.kernel,
        out_type=jax.ShapeDtypeStruct((nrows, ncols), jnp.float32),
        mesh=mesh,
        compiler_params=pltpu.CompilerParams(needs_layout_passes=False),
        scratch_types=[
            pltpu.VMEM((32,), jnp.float32),
            pltpu.VMEM((32,), jnp.float32),
            pltpu.VMEM((32,), jnp.float32),
            pltpu.VMEM((2, rows_per_chunk, ncols), jnp.float32),
            pltpu.VMEM((2, rows_per_chunk, ncols), jnp.int32),
            pltpu.VMEM((2, rows_per_chunk, ncols), jnp.float32),
            pltpu.SemaphoreType.DMA,
            pltpu.SemaphoreType.DMA,
            pltpu.SemaphoreType.DMA,
            pltpu.SemaphoreType.DMA,
            pltpu.SemaphoreType.DMA,
            pltpu.SemaphoreType.DMA,
        ],
    )
    def body(x_hbm, h_hbm, pat_hbm, zl_hbm, out_hbm,
             pat_v, zl_v, tab_v, x_v, h_v, o_v,
             sx0, sx1, sh0, sh1, so0, so1):
        wid = lax.axis_index("s") * 2 + lax.axis_index("c")
        base = wid * per_worker
        sx = (sx0, sx1)
        sh = (sh0, sh1)
        so = (so0, so1)

        pltpu.sync_copy(pat_hbm, pat_v)
        pltpu.sync_copy(zl_hbm, zl_v)
        for j in range(2):
            sl = pl.ds(j * _LANES, _LANES)
            p = pat_v[sl]
            z = zl_v[sl]
            sp = _softplus_vec(p)
            one_minus_sig = 1.0 / (1.0 + jnp.exp(z))
            tab_v[sl] = sp * one_minus_sig

        def start_in(c):
            b = c % 2
            rows = pl.ds(base + c * rows_per_chunk, rows_per_chunk)
            cx = pltpu.make_async_copy(x_hbm.at[rows], x_v.at[b], sx[b])
            cx.start()
            chh = pltpu.make_async_copy(h_hbm.at[rows], h_v.at[b], sh[b])
            chh.start()
            return cx, chh

        in_copies = [None, None]
        out_copies = [None, None]
        in_copies[0] = start_in(0)
        for c in range(n_chunks):
            b = c % 2
            if c + 1 < n_chunks:
                in_copies[(c + 1) % 2] = start_in(c + 1)
            cx, chh = in_copies[b]
            cx.wait()
            chh.wait()
            if out_copies[b] is not None:
                out_copies[b].wait()

            @plsc.parallel_loop(0, rows_per_chunk, step=1, unroll=2)
            def _(r):
                for cs in col_starts:
                    sl = pl.ds(cs, _LANES)
                    mv = plsc.load_gather(tab_v, [h_v[b, r, sl]])
                    o_v[b, r, sl] = x_v[b, r, sl] * mv

            co = pltpu.make_async_copy(
                o_v.at[b],
                out_hbm.at[pl.ds(base + c * rows_per_chunk, rows_per_chunk)],
                so[b])
            co.start()
            out_copies[b] = co
        for b in range(2):
            if out_copies[b] is not None:
                out_copies[b].wait()

    return body


def kernel(x, hour_indices, pattern, zero_logits):
    nrows, ncols = x.shape
    hf = hour_indices.astype(jnp.int32)
    pat = jnp.pad(pattern.astype(jnp.float32), (0, 8))
    zl = jnp.pad(zero_logits.astype(jnp.float32), (0, 8))
    return _sc_call(nrows, ncols, 64)(x, hf, pat, zl)


# speedup vs baseline: 4.2315x; 1.7781x over previous
import functools

import jax
import jax.numpy as jnp
from jax import lax
from jax.experimental import pallas as pl
from jax.experimental.pallas import tpu as pltpu
from jax.experimental.pallas import tpu_sc as plsc

_NUM_WORKERS = 32
_LANES = 16


def _softplus_vec(p):
    u = jnp.exp(-jnp.abs(p))
    w = 1.0 + u
    y = 2.0 * u / (2.0 + u)
    y = y + w * jnp.exp(-y) - 1.0
    y = y + w * jnp.exp(-y) - 1.0
    y = y + w * jnp.exp(-y) - 1.0
    return jnp.maximum(p, 0.0) + y


@functools.lru_cache(maxsize=None)
def _sc_call(nrows, ncols, rows_per_chunk):
    cols_per_worker = ncols // _NUM_WORKERS
    n_chunks = nrows // rows_per_chunk
    assert ncols % _NUM_WORKERS == 0 and nrows % rows_per_chunk == 0
    assert cols_per_worker % _LANES == 0 and rows_per_chunk % 8 == 0
    n_col_slices = cols_per_worker // _LANES
    mesh = plsc.VectorSubcoreMesh(core_axis_name="c", subcore_axis_name="s")

    @functools.partial(
        pl.kernel,
        out_type=jax.ShapeDtypeStruct((nrows, ncols), jnp.float32),
        mesh=mesh,
        compiler_params=pltpu.CompilerParams(needs_layout_passes=False),
        scratch_types=[
            pltpu.VMEM((32,), jnp.float32),
            pltpu.VMEM((32,), jnp.float32),
            pltpu.VMEM((32,), jnp.float32),
            pltpu.VMEM((2, rows_per_chunk, cols_per_worker), jnp.float32),
            pltpu.VMEM((2, rows_per_chunk, cols_per_worker), jnp.int32),
            pltpu.VMEM((2, rows_per_chunk, cols_per_worker), jnp.float32),
            pltpu.SemaphoreType.DMA,
            pltpu.SemaphoreType.DMA,
            pltpu.SemaphoreType.DMA,
            pltpu.SemaphoreType.DMA,
            pltpu.SemaphoreType.DMA,
            pltpu.SemaphoreType.DMA,
        ],
    )
    def body(x_hbm, h_hbm, pat_hbm, zl_hbm, out_hbm,
             pat_v, zl_v, tab_v, x_v, h_v, o_v,
             sx0, sx1, sh0, sh1, so0, so1):
        wid = lax.axis_index("s") * 2 + lax.axis_index("c")
        col0 = wid * cols_per_worker
        cols = pl.ds(col0, cols_per_worker)
        sx = (sx0, sx1)
        sh = (sh0, sh1)
        so = (so0, so1)

        pltpu.sync_copy(pat_hbm, pat_v)
        pltpu.sync_copy(zl_hbm, zl_v)
        for j in range(2):
            sl = pl.ds(j * _LANES, _LANES)
            p = pat_v[sl]
            z = zl_v[sl]
            sp = _softplus_vec(p)
            one_minus_sig = 1.0 / (1.0 + jnp.exp(z))
            tab_v[sl] = sp * one_minus_sig

        def start_in(c):
            b = c % 2
            rows = pl.ds(c * rows_per_chunk, rows_per_chunk)
            cx = pltpu.make_async_copy(x_hbm.at[rows, cols], x_v.at[b], sx[b])
            cx.start()
            chh = pltpu.make_async_copy(h_hbm.at[rows, cols], h_v.at[b], sh[b])
            chh.start()
            return cx, chh

        in_copies = [None, None]
        out_copies = [None, None]
        in_copies[0] = start_in(0)
        for c in range(n_chunks):
            b = c % 2
            if c + 1 < n_chunks:
                in_copies[(c + 1) % 2] = start_in(c + 1)
            cx, chh = in_copies[b]
            cx.wait()
            chh.wait()
            if out_copies[b] is not None:
                out_copies[b].wait()

            @plsc.parallel_loop(0, rows_per_chunk, step=1, unroll=2)
            def _(r):
                for k in range(n_col_slices):
                    sl = pl.ds(k * _LANES, _LANES)
                    mv = plsc.load_gather(tab_v, [h_v[b, r, sl]])
                    o_v[b, r, sl] = x_v[b, r, sl] * mv

            co = pltpu.make_async_copy(
                o_v.at[b],
                out_hbm.at[pl.ds(c * rows_per_chunk, rows_per_chunk), cols],
                so[b])
            co.start()
            out_copies[b] = co
        for b in range(2):
            if out_copies[b] is not None:
                out_copies[b].wait()

    return body


def kernel(x, hour_indices, pattern, zero_logits):
    nrows, ncols = x.shape
    xt = x.T
    ht = hour_indices.T.astype(jnp.int32)
    pat = jnp.pad(pattern.astype(jnp.float32), (0, 8))
    zl = jnp.pad(zero_logits.astype(jnp.float32), (0, 8))
    out_t = _sc_call(ncols, nrows, 40)(xt, ht, pat, zl)
    return out_t.T


# speedup vs baseline: 4.5654x; 1.0789x over previous
import functools

import jax
import jax.numpy as jnp
from jax import lax
from jax.experimental import pallas as pl
from jax.experimental.pallas import tpu as pltpu
from jax.experimental.pallas import tpu_sc as plsc

_NUM_WORKERS = 32
_LANES = 16


def _softplus_vec(p):
    u = jnp.exp(-jnp.abs(p))
    w = 1.0 + u
    y = 2.0 * u / (2.0 + u)
    y = y + w * jnp.exp(-y) - 1.0
    y = y + w * jnp.exp(-y) - 1.0
    y = y + w * jnp.exp(-y) - 1.0
    return jnp.maximum(p, 0.0) + y


@functools.lru_cache(maxsize=None)
def _sc_call(nrows, ncols, rows_per_chunk):
    cols_per_worker = ncols // _NUM_WORKERS
    n_chunks = nrows // rows_per_chunk
    assert ncols % _NUM_WORKERS == 0 and nrows % rows_per_chunk == 0
    assert cols_per_worker % _LANES == 0 and rows_per_chunk % 8 == 0
    n_col_slices = cols_per_worker // _LANES
    mesh = plsc.VectorSubcoreMesh(core_axis_name="c", subcore_axis_name="s")

    @functools.partial(
        pl.kernel,
        out_type=jax.ShapeDtypeStruct((nrows, ncols), jnp.float32),
        mesh=mesh,
        compiler_params=pltpu.CompilerParams(needs_layout_passes=False),
        scratch_types=[
            pltpu.VMEM((32,), jnp.float32),
            pltpu.VMEM((32,), jnp.float32),
            pltpu.VMEM((32,), jnp.float32),
            pltpu.VMEM((2, rows_per_chunk, cols_per_worker), jnp.float32),
            pltpu.VMEM((2, rows_per_chunk, cols_per_worker), jnp.int32),
            pltpu.VMEM((2, rows_per_chunk, cols_per_worker), jnp.float32),
            pltpu.SemaphoreType.DMA,
            pltpu.SemaphoreType.DMA,
            pltpu.SemaphoreType.DMA,
            pltpu.SemaphoreType.DMA,
            pltpu.SemaphoreType.DMA,
            pltpu.SemaphoreType.DMA,
        ],
    )
    def body(x_hbm, h_hbm, pat_hbm, zl_hbm, out_hbm,
             pat_v, zl_v, tab_v, x_v, h_v, o_v,
             sx0, sx1, sh0, sh1, so0, so1):
        wid = lax.axis_index("s") * 2 + lax.axis_index("c")
        col0 = wid * cols_per_worker
        cols = pl.ds(col0, cols_per_worker)
        sx = (sx0, sx1)
        sh = (sh0, sh1)
        so = (so0, so1)

        def start_in(c):
            b = c % 2
            rows = pl.ds(c * rows_per_chunk, rows_per_chunk)
            cx = pltpu.make_async_copy(x_hbm.at[rows, cols], x_v.at[b], sx[b])
            cx.start()
            chh = pltpu.make_async_copy(h_hbm.at[rows, cols], h_v.at[b], sh[b])
            chh.start()
            return cx, chh

        in_copies = [None, None]
        out_copies = [None, None]
        in_copies[0] = start_in(0)

        pltpu.sync_copy(pat_hbm, pat_v.at[pl.ds(0, 24)])
        pltpu.sync_copy(zl_hbm, zl_v.at[pl.ds(0, 24)])
        for j in range(2):
            sl = pl.ds(j * _LANES, _LANES)
            p = pat_v[sl]
            z = zl_v[sl]
            sp = _softplus_vec(p)
            one_minus_sig = 1.0 / (1.0 + jnp.exp(z))
            tab_v[sl] = sp * one_minus_sig

        for c in range(n_chunks):
            b = c % 2
            if c + 1 < n_chunks:
                in_copies[(c + 1) % 2] = start_in(c + 1)
            cx, chh = in_copies[b]
            cx.wait()
            chh.wait()
            if out_copies[b] is not None:
                out_copies[b].wait()

            @plsc.parallel_loop(0, rows_per_chunk, step=1, unroll=2)
            def _(r):
                for k in range(n_col_slices):
                    sl = pl.ds(k * _LANES, _LANES)
                    mv = plsc.load_gather(tab_v, [h_v[b, r, sl]])
                    o_v[b, r, sl] = x_v[b, r, sl] * mv

            co = pltpu.make_async_copy(
                o_v.at[b],
                out_hbm.at[pl.ds(c * rows_per_chunk, rows_per_chunk), cols],
                so[b])
            co.start()
            out_copies[b] = co
        for b in range(2):
            if out_copies[b] is not None:
                out_copies[b].wait()

    return body


def kernel(x, hour_indices, pattern, zero_logits):
    nrows, ncols = x.shape
    xt = x.T
    ht = hour_indices.T.astype(jnp.int32)
    pat = pattern.astype(jnp.float32)
    zl = zero_logits.astype(jnp.float32)
    out_t = _sc_call(ncols, nrows, 40)(xt, ht, pat, zl)
    return out_t.T


# speedup vs baseline: 4.9259x; 1.0790x over previous
import functools

import jax
import jax.numpy as jnp
from jax import lax
from jax.experimental import pallas as pl
from jax.experimental.pallas import tpu as pltpu
from jax.experimental.pallas import tpu_sc as plsc

_NUM_WORKERS = 32
_LANES = 16


def _softplus_vec(p):
    u = jnp.exp(-jnp.abs(p))
    w = 1.0 + u
    y = 2.0 * u / (2.0 + u)
    y = y + w * jnp.exp(-y) - 1.0
    y = y + w * jnp.exp(-y) - 1.0
    y = y + w * jnp.exp(-y) - 1.0
    return jnp.maximum(p, 0.0) + y


@functools.lru_cache(maxsize=None)
def _sc_call(nrows, ncols, chunk_rows):
    cols_per_worker = ncols // _NUM_WORKERS
    assert ncols % _NUM_WORKERS == 0
    assert sum(chunk_rows) == nrows and all(r % 8 == 0 for r in chunk_rows)
    assert cols_per_worker % _LANES == 0
    max_rows = max(chunk_rows)
    row_starts = [sum(chunk_rows[:i]) for i in range(len(chunk_rows))]
    n_chunks = len(chunk_rows)
    n_col_slices = cols_per_worker // _LANES
    mesh = plsc.VectorSubcoreMesh(core_axis_name="c", subcore_axis_name="s")

    @functools.partial(
        pl.kernel,
        out_type=jax.ShapeDtypeStruct((nrows, ncols), jnp.float32),
        mesh=mesh,
        compiler_params=pltpu.CompilerParams(needs_layout_passes=False),
        scratch_types=[
            pltpu.VMEM((32,), jnp.float32),
            pltpu.VMEM((32,), jnp.float32),
            pltpu.VMEM((32,), jnp.float32),
            pltpu.VMEM((2, max_rows, cols_per_worker), jnp.float32),
            pltpu.VMEM((2, max_rows, cols_per_worker), jnp.int32),
            pltpu.VMEM((2, max_rows, cols_per_worker), jnp.float32),
            pltpu.SemaphoreType.DMA,
            pltpu.SemaphoreType.DMA,
            pltpu.SemaphoreType.DMA,
            pltpu.SemaphoreType.DMA,
            pltpu.SemaphoreType.DMA,
            pltpu.SemaphoreType.DMA,
        ],
    )
    def body(x_hbm, h_hbm, pat_hbm, zl_hbm, out_hbm,
             pat_v, zl_v, tab_v, x_v, h_v, o_v,
             sx0, sx1, sh0, sh1, so0, so1):
        wid = lax.axis_index("s") * 2 + lax.axis_index("c")
        col0 = wid * cols_per_worker
        cols = pl.ds(col0, cols_per_worker)
        sx = (sx0, sx1)
        sh = (sh0, sh1)
        so = (so0, so1)

        def start_in(c):
            b = c % 2
            rc = chunk_rows[c]
            rows = pl.ds(row_starts[c], rc)
            dst_rows = pl.ds(0, rc)
            cx = pltpu.make_async_copy(
                x_hbm.at[rows, cols], x_v.at[b, dst_rows], sx[b])
            cx.start()
            chh = pltpu.make_async_copy(
                h_hbm.at[rows, cols], h_v.at[b, dst_rows], sh[b])
            chh.start()
            return cx, chh

        in_copies = [None, None]
        out_copies = [None, None]
        in_copies[0] = start_in(0)

        cp = pltpu.make_async_copy(pat_hbm, pat_v.at[pl.ds(0, 24)], so0)
        cp.start()
        cz = pltpu.make_async_copy(zl_hbm, zl_v.at[pl.ds(0, 24)], so1)
        cz.start()
        cp.wait()
        cz.wait()
        for j in range(2):
            sl = pl.ds(j * _LANES, _LANES)
            p = pat_v[sl]
            z = zl_v[sl]
            sp = _softplus_vec(p)
            one_minus_sig = 1.0 / (1.0 + jnp.exp(z))
            tab_v[sl] = sp * one_minus_sig

        for c in range(n_chunks):
            b = c % 2
            rc = chunk_rows[c]
            if c + 1 < n_chunks:
                in_copies[(c + 1) % 2] = start_in(c + 1)
            cx, chh = in_copies[b]
            cx.wait()
            chh.wait()
            if out_copies[b] is not None:
                out_copies[b].wait()

            @plsc.parallel_loop(0, rc, step=1, unroll=1)
            def _(r):
                for k in range(n_col_slices):
                    sl = pl.ds(k * _LANES, _LANES)
                    mv = plsc.load_gather(tab_v, [h_v[b, r, sl]])
                    o_v[b, r, sl] = x_v[b, r, sl] * mv

            co = pltpu.make_async_copy(
                o_v.at[b, pl.ds(0, rc)],
                out_hbm.at[pl.ds(row_starts[c], rc), cols],
                so[b])
            co.start()
            out_copies[b] = co
        for b in range(2):
            if out_copies[b] is not None:
                out_copies[b].wait()

    return body


def kernel(x, hour_indices, pattern, zero_logits):
    nrows, ncols = x.shape
    xt = x.T
    ht = hour_indices.T.astype(jnp.int32)
    pat = pattern.astype(jnp.float32)
    zl = zero_logits.astype(jnp.float32)
    out_t = _sc_call(ncols, nrows, (8, 32, 40, 40, 40, 40))(xt, ht, pat, zl)
    return out_t.T


# speedup vs baseline: 5.6525x; 1.1475x over previous
import functools

import jax
import jax.numpy as jnp
from jax import lax
from jax.experimental import pallas as pl
from jax.experimental.pallas import tpu as pltpu
from jax.experimental.pallas import tpu_sc as plsc

_NUM_WORKERS = 32
_LANES = 16


def _softplus_vec(p):
    u = jnp.exp(-jnp.abs(p))
    w = 1.0 + u
    y = 2.0 * u / (2.0 + u)
    y = y + w * jnp.exp(-y) - 1.0
    y = y + w * jnp.exp(-y) - 1.0
    y = y + w * jnp.exp(-y) - 1.0
    return jnp.maximum(p, 0.0) + y


@functools.lru_cache(maxsize=None)
def _sc_call(nrows, ncols, chunk_rows):
    cols_per_worker = ncols // _NUM_WORKERS
    assert ncols % _NUM_WORKERS == 0
    assert sum(chunk_rows) == nrows and all(r % 8 == 0 for r in chunk_rows)
    assert cols_per_worker % _LANES == 0
    max_rows = max(chunk_rows)
    row_starts = [sum(chunk_rows[:i]) for i in range(len(chunk_rows))]
    n_chunks = len(chunk_rows)
    n_col_slices = cols_per_worker // _LANES
    mesh = plsc.VectorSubcoreMesh(core_axis_name="c", subcore_axis_name="s")

    @functools.partial(
        pl.kernel,
        out_type=jax.ShapeDtypeStruct((nrows, ncols), jnp.float32),
        mesh=mesh,
        compiler_params=pltpu.CompilerParams(needs_layout_passes=False),
        scratch_types=[
            pltpu.VMEM((32,), jnp.float32),
            pltpu.VMEM((32,), jnp.float32),
            pltpu.VMEM((32,), jnp.float32),
            pltpu.VMEM((2, max_rows, cols_per_worker), jnp.float32),
            pltpu.VMEM((2, max_rows, cols_per_worker), jnp.int32),
            pltpu.VMEM((2, max_rows, cols_per_worker), jnp.float32),
            pltpu.SemaphoreType.DMA,
            pltpu.SemaphoreType.DMA,
            pltpu.SemaphoreType.DMA,
            pltpu.SemaphoreType.DMA,
            pltpu.SemaphoreType.DMA,
            pltpu.SemaphoreType.DMA,
        ],
    )
    def body(x_hbm, h_hbm, pat_hbm, zl_hbm, out_hbm,
             pat_v, zl_v, tab_v, x_v, h_v, o_v,
             sx0, sx1, sh0, sh1, so0, so1):
        wid = lax.axis_index("s") * 2 + lax.axis_index("c")
        col0 = wid * cols_per_worker
        cols = pl.ds(col0, cols_per_worker)
        sx = (sx0, sx1)
        sh = (sh0, sh1)
        so = (so0, so1)

        def start_in(c):
            b = c % 2
            rc = chunk_rows[c]
            rows = pl.ds(row_starts[c], rc)
            dst_rows = pl.ds(0, rc)
            cx = pltpu.make_async_copy(
                x_hbm.at[rows, cols], x_v.at[b, dst_rows], sx[b])
            cx.start()
            chh = pltpu.make_async_copy(
                h_hbm.at[rows, cols], h_v.at[b, dst_rows], sh[b])
            chh.start()
            return cx, chh

        in_copies = [None, None]
        out_copies = [None, None]
        in_copies[0] = start_in(0)

        cp = pltpu.make_async_copy(pat_hbm, pat_v.at[pl.ds(0, 24)], so0)
        cp.start()
        cz = pltpu.make_async_copy(zl_hbm, zl_v.at[pl.ds(0, 24)], so1)
        cz.start()
        cp.wait()
        cz.wait()
        for j in range(2):
            sl = pl.ds(j * _LANES, _LANES)
            p = pat_v[sl]
            z = zl_v[sl]
            sp = _softplus_vec(p)
            one_minus_sig = 1.0 / (1.0 + jnp.exp(z))
            tab_v[sl] = sp * one_minus_sig

        for c in range(n_chunks):
            b = c % 2
            rc = chunk_rows[c]
            if c + 1 < n_chunks:
                in_copies[(c + 1) % 2] = start_in(c + 1)
            cx, chh = in_copies[b]
            cx.wait()
            chh.wait()
            if out_copies[b] is not None:
                out_copies[b].wait()

            @plsc.parallel_loop(0, rc, step=1, unroll=1)
            def _(r):
                @plsc.parallel_loop(0, cols_per_worker, step=_LANES, unroll=8)
                def _(s):
                    sl = pl.ds(s, _LANES)
                    mv = plsc.load_gather(tab_v, [h_v[b, r, sl]])
                    o_v[b, r, sl] = x_v[b, r, sl] * mv

            co = pltpu.make_async_copy(
                o_v.at[b, pl.ds(0, rc)],
                out_hbm.at[pl.ds(row_starts[c], rc), cols],
                so[b])
            co.start()
            out_copies[b] = co
        for b in range(2):
            if out_copies[b] is not None:
                out_copies[b].wait()

    return body


def kernel(x, hour_indices, pattern, zero_logits):
    nrows, ncols = x.shape
    xt = x.T
    ht = hour_indices.T.astype(jnp.int32)
    pat = pattern.astype(jnp.float32)
    zl = zero_logits.astype(jnp.float32)
    out_t = _sc_call(ncols, nrows, (8, 32, 40, 40, 40, 40))(xt, ht, pat, zl)
    return out_t.T


# speedup vs baseline: 5.8590x; 1.0365x over previous
import functools

import jax
import jax.numpy as jnp
from jax import lax
from jax.experimental import pallas as pl
from jax.experimental.pallas import tpu as pltpu
from jax.experimental.pallas import tpu_sc as plsc

_NUM_WORKERS = 32
_LANES = 16
_CP = 8
_CL = 32
_NPAIRS = 3


def _softplus_vec(p):
    u = jnp.exp(-jnp.abs(p))
    w = 1.0 + u
    y = 2.0 * u / (2.0 + u)
    y = y + w * jnp.exp(-y) - 1.0
    y = y + w * jnp.exp(-y) - 1.0
    y = y + w * jnp.exp(-y) - 1.0
    return jnp.maximum(p, 0.0) + y


@functools.lru_cache(maxsize=None)
def _sc_call(nrows, ncols):
    cols_per_worker = ncols // _NUM_WORKERS
    assert ncols % _NUM_WORKERS == 0
    assert nrows == _CP + 2 * _NPAIRS * _CL
    assert cols_per_worker % _LANES == 0
    mesh = plsc.VectorSubcoreMesh(core_axis_name="c", subcore_axis_name="s")

    @functools.partial(
        pl.kernel,
        out_type=jax.ShapeDtypeStruct((nrows, ncols), jnp.float32),
        mesh=mesh,
        compiler_params=pltpu.CompilerParams(needs_layout_passes=False),
        scratch_types=[
            pltpu.VMEM((32,), jnp.float32),
            pltpu.VMEM((32,), jnp.float32),
            pltpu.VMEM((32,), jnp.float32),
            pltpu.VMEM((2, _CL, cols_per_worker), jnp.float32),
            pltpu.VMEM((2, _CL, cols_per_worker), jnp.int32),
            pltpu.VMEM((2, _CL, cols_per_worker), jnp.float32),
            pltpu.SemaphoreType.DMA,
            pltpu.SemaphoreType.DMA,
            pltpu.SemaphoreType.DMA,
            pltpu.SemaphoreType.DMA,
            pltpu.SemaphoreType.DMA,
            pltpu.SemaphoreType.DMA,
            pltpu.SemaphoreType.DMA,
        ],
    )
    def body(x_hbm, h_hbm, pat_hbm, zl_hbm, out_hbm,
             pat_v, zl_v, tab_v, x_v, h_v, o_v,
             sx0, sx1, sh0, sh1, so0, so1, sp):
        wid = lax.axis_index("s") * 2 + lax.axis_index("c")
        cols = pl.ds(wid * cols_per_worker, cols_per_worker)

        def in_start(row0, rc, b, dst_rows, semx, semh):
            cx = pltpu.make_async_copy(
                x_hbm.at[pl.ds(row0, rc), cols], x_v.at[b, dst_rows], semx)
            cx.start()
            chh = pltpu.make_async_copy(
                h_hbm.at[pl.ds(row0, rc), cols], h_v.at[b, dst_rows], semh)
            chh.start()
            return cx, chh

        def in_wait(row0, rc, b, dst_rows, semx, semh):
            pltpu.make_async_copy(
                x_hbm.at[pl.ds(row0, rc), cols], x_v.at[b, dst_rows], semx).wait()
            pltpu.make_async_copy(
                h_hbm.at[pl.ds(row0, rc), cols], h_v.at[b, dst_rows], semh).wait()

        def out_start(row0, rc, b, src_rows, sem):
            pltpu.make_async_copy(
                o_v.at[b, src_rows], out_hbm.at[pl.ds(row0, rc), cols], sem
            ).start()

        def out_wait(row0, rc, b, src_rows, sem):
            pltpu.make_async_copy(
                o_v.at[b, src_rows], out_hbm.at[pl.ds(row0, rc), cols], sem
            ).wait()

        def compute(b, rc):
            @plsc.parallel_loop(0, rc, step=1, unroll=1)
            def _(r):
                @plsc.parallel_loop(0, cols_per_worker, step=_LANES, unroll=8)
                def _(s):
                    sl = pl.ds(s, _LANES)
                    mv = plsc.load_gather(tab_v, [h_v[b, r, sl]])
                    o_v[b, r, sl] = x_v[b, r, sl] * mv

        p_rows = pl.ds(0, _CP)
        full = pl.ds(0, _CL)

        in_start(0, _CP, 0, p_rows, sx0, sh0)
        in_start(_CP, _CL, 1, full, sx1, sh1)

        cp_ = pltpu.make_async_copy(pat_hbm, pat_v.at[pl.ds(0, 24)], sp)
        cp_.start()
        cz = pltpu.make_async_copy(zl_hbm, zl_v.at[pl.ds(0, 24)], sp)
        cz.start()
        cp_.wait()
        cz.wait()
        for j in range(2):
            sl = pl.ds(j * _LANES, _LANES)
            sp_v = _softplus_vec(pat_v[sl])
            one_minus_sig = 1.0 / (1.0 + jnp.exp(zl_v[sl]))
            tab_v[sl] = sp_v * one_minus_sig

        in_wait(0, _CP, 0, p_rows, sx0, sh0)
        compute(0, _CP)
        out_start(0, _CP, 0, p_rows, sp)

        def pair(i, carry):
            r1 = _CP + 2 * i * _CL
            r2 = r1 + _CL
            in_start(r2, _CL, 0, full, sx0, sh0)
            in_wait(r1, _CL, 1, full, sx1, sh1)

            @pl.when(i > 0)
            def _():
                out_wait(r1 - 2 * _CL, _CL, 1, full, so1)

            compute(1, _CL)
            out_start(r1, _CL, 1, full, so1)

            @pl.when(i < _NPAIRS - 1)
            def _():
                in_start(r2 + _CL, _CL, 1, full, sx1, sh1)

            in_wait(r2, _CL, 0, full, sx0, sh0)

            @pl.when(i == 0)
            def _():
                out_wait(0, _CP, 0, p_rows, sp)

            @pl.when(i > 0)
            def _():
                out_wait(r2 - 2 * _CL, _CL, 0, full, so0)

            compute(0, _CL)
            out_start(r2, _CL, 0, full, so0)
            return carry

        lax.fori_loop(0, _NPAIRS, pair, 0)

        last = _CP + (2 * _NPAIRS - 1) * _CL
        out_wait(last - _CL, _CL, 1, full, so1)
        out_wait(last, _CL, 0, full, so0)

    return body


def kernel(x, hour_indices, pattern, zero_logits):
    nrows, ncols = x.shape
    xt = x.T
    ht = hour_indices.T.astype(jnp.int32)
    pat = pattern.astype(jnp.float32)
    zl = zero_logits.astype(jnp.float32)
    out_t = _sc_call(ncols, nrows)(xt, ht, pat, zl)
    return out_t.T
